# baseline (speedup 1.0000x reference)
import functools
import math

import jax
import jax.numpy as jnp
from jax import lax
from jax.experimental import pallas as pl
from jax.experimental.pallas import tpu as pltpu
from jax.experimental.pallas import tpu_sc as plsc

D_MODEL = 1024
SCALE = math.sqrt(D_MODEL)
LANES = 16
CHUNK = 16
NBUF = 6
AHEAD = 2
N_WORKERS = 32


@functools.partial(jax.jit, static_argnums=(2,))
def _embed_lookup(x, table, n_total):
    n_per_w = n_total // N_WORKERS
    n_chunks = n_per_w // CHUNK
    s = x.shape[1]
    w_per_row = s // n_per_w
    mesh = plsc.VectorSubcoreMesh(core_axis_name="c", subcore_axis_name="s")

    @functools.partial(
        pl.kernel,
        mesh=mesh,
        out_type=jax.ShapeDtypeStruct((n_total, D_MODEL), jnp.float32),
        scratch_types=[
            pltpu.VMEM((n_per_w + AHEAD * CHUNK,), jnp.int32),
            pltpu.VMEM((NBUF, CHUNK, D_MODEL), jnp.float32),
            pltpu.SemaphoreType.DMA((NBUF,)),
            pltpu.SemaphoreType.DMA((NBUF,)),
        ],
    )
    def k(x_hbm, table_hbm, out_hbm, idx_v, bufs, in_sem, out_sem):
        num_c = 2
        wid = lax.axis_index("s") * num_c + lax.axis_index("c")
        base = wid * n_per_w
        pltpu.sync_copy(
            x_hbm.at[wid // w_per_row, pl.ds((wid % w_per_row) * n_per_w, n_per_w)],
            idx_v.at[pl.ds(0, n_per_w)],
        )

        def gather(c, b):
            pltpu.async_copy(
                table_hbm.at[idx_v.at[pl.ds(c * CHUNK, CHUNK)]],
                bufs.at[b],
                in_sem.at[b],
            )

        def wait_in(b):
            pltpu.make_async_copy(
                table_hbm.at[idx_v.at[pl.ds(0, CHUNK)]], bufs.at[b], in_sem.at[b]
            ).wait()

        def wait_out(b):
            pltpu.make_async_copy(
                bufs.at[b], out_hbm.at[pl.ds(base, CHUNK)], out_sem.at[b]
            ).wait()

        for b in range(AHEAD):
            gather(b, b)
        for t in range(AHEAD * CHUNK // LANES):
            idx_v[pl.ds(n_per_w + t * LANES, LANES)] = lax.iota(
                jnp.int32, LANES
            ) + (wid * AHEAD * CHUNK + t * LANES)
        for b in range(AHEAD, NBUF):
            pltpu.async_copy(
                bufs.at[b],
                out_hbm.at[pl.ds(base + (n_chunks - NBUF + b) * CHUNK, CHUNK)],
                out_sem.at[b],
            )

        def emit_chunk(c, b):
            bn = (b + AHEAD) % NBUF
            wait_out(bn)
            gather(c + AHEAD, bn)
            wait_in(b)

            @plsc.parallel_loop(0, CHUNK, unroll=1)
            def row_body(r):
                for j in range(D_MODEL // LANES):
                    sl = pl.ds(j * LANES, LANES)
                    bufs[b, r, sl] = bufs[b, r, sl] * SCALE

            pltpu.async_copy(
                bufs.at[b],
                out_hbm.at[pl.ds(base + c * CHUNK, CHUNK)],
                out_sem.at[b],
            )

        n_full = (n_chunks // NBUF) * NBUF

        def group_body(i, carry):
            for b in range(NBUF):
                emit_chunk(i * NBUF + b, b)
            return carry

        lax.fori_loop(0, n_chunks // NBUF, group_body, 0)
        for c in range(n_full, n_chunks):
            emit_chunk(c, c % NBUF)
        for t in range(AHEAD):
            wait_in((n_chunks + t) % NBUF)
        for t in range(NBUF - AHEAD):
            wait_out((n_chunks - (NBUF - AHEAD) + t) % NBUF)

    return k(x, table)


def kernel(x, table):
    b, s = x.shape
    n_total = b * s
    out = _embed_lookup(x, table, n_total)
    return out.reshape(b, s, D_MODEL)


# speedup vs baseline: 1.0129x; 1.0129x over previous
import functools
import math

import jax
import jax.numpy as jnp
from jax import lax
from jax.experimental import pallas as pl
from jax.experimental.pallas import tpu as pltpu
from jax.experimental.pallas import tpu_sc as plsc

D_MODEL = 1024
SCALE = math.sqrt(D_MODEL)
LANES = 16
CHUNK = 16
NBUF = 4
AHEAD = 2
N_WORKERS = 32


@functools.partial(jax.jit, static_argnums=(2,))
def _embed_lookup(x, table, n_total):
    n_per_w = n_total // N_WORKERS
    n_chunks = n_per_w // CHUNK
    s = x.shape[1]
    w_per_row = s // n_per_w
    mesh = plsc.VectorSubcoreMesh(core_axis_name="c", subcore_axis_name="s")

    @functools.partial(
        pl.kernel,
        mesh=mesh,
        out_type=jax.ShapeDtypeStruct((n_total, D_MODEL), jnp.float32),
        scratch_types=[
            pltpu.VMEM((n_per_w + AHEAD * CHUNK,), jnp.int32),
            pltpu.VMEM((NBUF, CHUNK, D_MODEL), jnp.float32),
            pltpu.SemaphoreType.DMA((NBUF,)),
            pltpu.SemaphoreType.DMA((NBUF,)),
        ],
    )
    def k(x_hbm, table_hbm, out_hbm, idx_v, bufs, in_sem, out_sem):
        num_c = 2
        wid = lax.axis_index("s") * num_c + lax.axis_index("c")
        base = wid * n_per_w
        pltpu.sync_copy(
            x_hbm.at[wid // w_per_row, pl.ds((wid % w_per_row) * n_per_w, n_per_w)],
            idx_v.at[pl.ds(0, n_per_w)],
        )

        def gather(c, b):
            pltpu.async_copy(
                table_hbm.at[idx_v.at[pl.ds(c * CHUNK, CHUNK)]],
                bufs.at[b],
                in_sem.at[b],
            )

        def wait_in(b):
            pltpu.make_async_copy(
                table_hbm.at[idx_v.at[pl.ds(0, CHUNK)]], bufs.at[b], in_sem.at[b]
            ).wait()

        def wait_out(b):
            pltpu.make_async_copy(
                bufs.at[b], out_hbm.at[pl.ds(base, CHUNK)], out_sem.at[b]
            ).wait()

        for b in range(AHEAD):
            gather(b, b)
        for t in range(AHEAD * CHUNK // LANES):
            idx_v[pl.ds(n_per_w + t * LANES, LANES)] = lax.iota(
                jnp.int32, LANES
            ) + (wid * AHEAD * CHUNK + t * LANES)
        for b in range(AHEAD, NBUF):
            pltpu.async_copy(
                bufs.at[b],
                out_hbm.at[pl.ds(base + (n_chunks - NBUF + b) * CHUNK, CHUNK)],
                out_sem.at[b],
            )

        def emit_chunk(c, b):
            bn = (b + AHEAD) % NBUF
            wait_out(bn)
            gather(c + AHEAD, bn)
            wait_in(b)
            half = CHUNK // 2
            for h in range(2):

                @plsc.parallel_loop(h * half, (h + 1) * half, unroll=1)
                def row_body(r):
                    for j in range(D_MODEL // LANES):
                        sl = pl.ds(j * LANES, LANES)
                        bufs[b, r, sl] = bufs[b, r, sl] * SCALE

                pltpu.async_copy(
                    bufs.at[b].at[pl.ds(h * half, half)],
                    out_hbm.at[pl.ds(base + c * CHUNK + h * half, half)],
                    out_sem.at[b],
                )

        n_full = (n_chunks // NBUF) * NBUF

        def group_body(i, carry):
            for b in range(NBUF):
                emit_chunk(i * NBUF + b, b)
            return carry

        lax.fori_loop(0, n_chunks // NBUF, group_body, 0)
        for c in range(n_full, n_chunks):
            emit_chunk(c, c % NBUF)
        for t in range(AHEAD):
            wait_in((n_chunks + t) % NBUF)
        for t in range(NBUF - AHEAD):
            wait_out((n_chunks - (NBUF - AHEAD) + t) % NBUF)

    return k(x, table)


def kernel(x, table):
    b, s = x.shape
    n_total = b * s
    out = _embed_lookup(x, table, n_total)
    return out.reshape(b, s, D_MODEL)


# speedup vs baseline: 1.0439x; 1.0306x over previous
import functools
import math

import jax
import jax.numpy as jnp
from jax import lax
from jax.experimental import pallas as pl
from jax.experimental.pallas import tpu as pltpu
from jax.experimental.pallas import tpu_sc as plsc

D_MODEL = 1024
SCALE = math.sqrt(D_MODEL)
LANES = 16
CHUNK = 16
NBUF = 4
AHEAD = 2
N_WORKERS = 32


@functools.partial(jax.jit, static_argnums=(2,))
def _embed_lookup(x, table, n_total):
    n_per_w = n_total // N_WORKERS
    n_chunks = n_per_w // CHUNK
    s = x.shape[1]
    w_per_row = s // n_per_w
    mesh = plsc.VectorSubcoreMesh(core_axis_name="c", subcore_axis_name="s")

    @functools.partial(
        pl.kernel,
        mesh=mesh,
        out_type=jax.ShapeDtypeStruct((n_total, D_MODEL), jnp.float32),
        scratch_types=[
            pltpu.VMEM((n_per_w + AHEAD * CHUNK,), jnp.int32),
            pltpu.VMEM((NBUF, CHUNK, D_MODEL), jnp.float32),
            pltpu.SemaphoreType.DMA((NBUF,)),
            pltpu.SemaphoreType.DMA((NBUF,)),
        ],
    )
    def k(x_hbm, table_hbm, out_hbm, idx_v, bufs, in_sem, out_sem):
        num_c = 2
        wid = lax.axis_index("s") * num_c + lax.axis_index("c")
        base = wid * n_per_w
        pltpu.sync_copy(
            x_hbm.at[wid // w_per_row, pl.ds((wid % w_per_row) * n_per_w, n_per_w)],
            idx_v.at[pl.ds(0, n_per_w)],
        )

        def gather(c, b):
            pltpu.async_copy(
                table_hbm.at[idx_v.at[pl.ds(c * CHUNK, CHUNK)]],
                bufs.at[b],
                in_sem.at[b],
            )

        def wait_in(b):
            pltpu.make_async_copy(
                table_hbm.at[idx_v.at[pl.ds(0, CHUNK)]], bufs.at[b], in_sem.at[b]
            ).wait()

        def wait_out(b):
            pltpu.make_async_copy(
                bufs.at[b], out_hbm.at[pl.ds(base, CHUNK)], out_sem.at[b]
            ).wait()

        for b in range(AHEAD):
            gather(b, b)
        for t in range(AHEAD * CHUNK // LANES):
            idx_v[pl.ds(n_per_w + t * LANES, LANES)] = lax.iota(
                jnp.int32, LANES
            ) + (wid * AHEAD * CHUNK + t * LANES)
        for b in range(AHEAD, NBUF):
            pltpu.async_copy(
                bufs.at[b],
                out_hbm.at[pl.ds(base + (n_chunks - NBUF + b) * CHUNK, CHUNK)],
                out_sem.at[b],
            )

        def emit_chunk(c, b):
            bn = (b + AHEAD) % NBUF
            wait_out(bn)
            gather(c + AHEAD, bn)
            wait_in(b)

            @plsc.parallel_loop(0, CHUNK, unroll=1)
            def row_body(r):
                for j in range(D_MODEL // LANES):
                    sl = pl.ds(j * LANES, LANES)
                    bufs[b, r, sl] = bufs[b, r, sl] * SCALE

            pltpu.async_copy(
                bufs.at[b],
                out_hbm.at[pl.ds(base + c * CHUNK, CHUNK)],
                out_sem.at[b],
            )

        n_full = (n_chunks // NBUF) * NBUF

        def group_body(i, carry):
            for b in range(NBUF):
                emit_chunk(i * NBUF + b, b)
            return carry

        lax.fori_loop(0, n_chunks // NBUF, group_body, 0)
        for c in range(n_full, n_chunks):
            emit_chunk(c, c % NBUF)
        for t in range(AHEAD):
            wait_in((n_chunks + t) % NBUF)
        for t in range(NBUF - AHEAD):
            wait_out((n_chunks - (NBUF - AHEAD) + t) % NBUF)

    return k(x, table)


def kernel(x, table):
    b, s = x.shape
    n_total = b * s
    out = _embed_lookup(x, table, n_total)
    return out.reshape(b, s, D_MODEL)


# speedup vs baseline: 1.0456x; 1.0016x over previous
import functools
import math

import jax
import jax.numpy as jnp
from jax import lax
from jax.experimental import pallas as pl
from jax.experimental.pallas import tpu as pltpu
from jax.experimental.pallas import tpu_sc as plsc

D_MODEL = 1024
SCALE = math.sqrt(D_MODEL)
LANES = 16
CHUNK = 16
NBUF = 4
AHEAD = 2
N_WORKERS = 32


@functools.partial(jax.jit, static_argnums=(2,))
def _embed_lookup(x, table, n_total):
    n_per_w = n_total // N_WORKERS
    n_chunks = n_per_w // CHUNK
    s = x.shape[1]
    w_per_row = s // n_per_w
    mesh = plsc.VectorSubcoreMesh(core_axis_name="c", subcore_axis_name="s")

    @functools.partial(
        pl.kernel,
        mesh=mesh,
        out_type=jax.ShapeDtypeStruct((n_total, D_MODEL), jnp.float32),
        scratch_types=[
            pltpu.VMEM((n_per_w + AHEAD * CHUNK,), jnp.int32),
            pltpu.VMEM((NBUF, CHUNK, D_MODEL), jnp.float32),
            pltpu.SemaphoreType.DMA((NBUF,)),
            pltpu.SemaphoreType.DMA((NBUF,)),
        ],
    )
    def k(x_hbm, table_hbm, out_hbm, idx_v, bufs, in_sem, out_sem):
        num_c = 2
        wid = lax.axis_index("s") * num_c + lax.axis_index("c")
        base = wid * n_per_w
        pltpu.sync_copy(
            x_hbm.at[wid // w_per_row, pl.ds((wid % w_per_row) * n_per_w, n_per_w)],
            idx_v.at[pl.ds(0, n_per_w)],
        )

        def gather(c, b):
            pltpu.async_copy(
                table_hbm.at[idx_v.at[pl.ds(c * CHUNK, CHUNK)]],
                bufs.at[b],
                in_sem.at[b],
            )

        def wait_in(b):
            pltpu.make_async_copy(
                table_hbm.at[idx_v.at[pl.ds(0, CHUNK)]], bufs.at[b], in_sem.at[b]
            ).wait()

        def wait_out(b):
            pltpu.make_async_copy(
                bufs.at[b], out_hbm.at[pl.ds(base, CHUNK)], out_sem.at[b]
            ).wait()

        for b in range(AHEAD):
            gather(b, b)
        for t in range(AHEAD * CHUNK // LANES):
            idx_v[pl.ds(n_per_w + t * LANES, LANES)] = lax.iota(
                jnp.int32, LANES
            ) + (wid * AHEAD * CHUNK + t * LANES)
        for b in range(AHEAD, NBUF):
            pltpu.async_copy(
                bufs.at[b],
                out_hbm.at[pl.ds(base + (n_chunks - NBUF + b) * CHUNK, CHUNK)],
                out_sem.at[b],
            )

        def emit_chunk(c, b):
            bn = (b + AHEAD) % NBUF
            wait_out(bn)
            gather(c + AHEAD, bn)
            wait_in(b)

            @plsc.parallel_loop(0, CHUNK, unroll=1)
            def row_body(r):
                for j in range(D_MODEL // LANES):
                    sl = pl.ds(j * LANES, LANES)
                    bufs[b, r, sl] = bufs[b, r, sl] * SCALE

            pltpu.async_copy(
                bufs.at[b],
                out_hbm.at[pl.ds(base + c * CHUNK, CHUNK)],
                out_sem.at[b],
            )

        n_full = (n_chunks // NBUF) * NBUF

        def group_body(i, carry):
            for b in range(NBUF):
                emit_chunk(i * NBUF + b, b)
            return carry

        lax.fori_loop(0, n_chunks // NBUF, group_body, 0)
        for c in range(n_full, n_chunks):
            emit_chunk(c, c % NBUF)
        for t in range(AHEAD):
            wait_in((n_chunks + t) % NBUF)
        for t in range(NBUF - AHEAD):
            wait_out((n_chunks - (NBUF - AHEAD) + t) % NBUF)

    return k(x, table)


def kernel(x, table):
    b, s = x.shape
    n_total = b * s
    out = _embed_lookup(x.astype(jnp.int32), table, n_total)
    return out.reshape(b, s, D_MODEL)
